# Initial kernel scaffold; baseline (speedup 1.0000x reference)
import functools

import jax
import jax.numpy as jnp
from jax import lax
from jax.experimental import pallas as pl
from jax.experimental.pallas import tpu as pltpu
from jax.experimental.pallas import tpu_sc as plsc

N = 10000
D = 128
DO = 2
E = 320000

NC = 2
NS = 16
NW = NC * NS
L = 16

EH_T = E // NW

CHUNK = 128
K_T = 80
E_T = CHUNK * K_T
E_PAD = E_T * NW
N_PAD = 10240
ZROWS = N_PAD // NS
DUMMY = N_PAD - 1

_mesh = plsc.VectorSubcoreMesh(
    core_axis_name="c", subcore_axis_name="s", num_cores=NC, num_subcores=NS
)


@functools.partial(
    pl.kernel,
    out_type=jax.ShapeDtypeStruct((NW, N), jnp.float32),
    mesh=_mesh,
    scratch_types=[
        pltpu.VMEM((EH_T,), jnp.int32),
        pltpu.VMEM((N,), jnp.float32),
    ],
)
def _sc_degree(col_hbm, deg_parts_hbm, colv, degv):
    c = lax.axis_index("c")
    s = lax.axis_index("s")
    wid = c * NS + s
    pltpu.sync_copy(col_hbm.at[pl.ds(wid * EH_T, EH_T)], colv)
    zero = jnp.zeros((L,), jnp.float32)

    def zbody(i, carry):
        degv[pl.ds(i * L, L)] = zero
        return carry

    lax.fori_loop(0, N // L, zbody, 0)
    ones = jnp.ones((L,), jnp.float32)

    def hbody(i, carry):
        idx = colv[pl.ds(i * L, L)]
        plsc.addupdate_scatter(degv, [idx], ones)
        return carry

    lax.fori_loop(0, EH_T // L, hbody, 0)
    pltpu.sync_copy(degv, deg_parts_hbm.at[wid])


@functools.partial(
    pl.kernel,
    out_type=jax.ShapeDtypeStruct((NC, N_PAD, D), jnp.float32),
    mesh=_mesh,
    scratch_types=[
        pltpu.VMEM((K_T, CHUNK), jnp.int32),
        pltpu.VMEM((K_T, CHUNK), jnp.int32),
        pltpu.VMEM((CHUNK, D), jnp.float32),
        pltpu.VMEM_SHARED((N_PAD, D), jnp.float32),
        pltpu.SemaphoreType.DMA,
    ],
)
def _sc_msg(g_hbm, row_hbm, col_hbm, out_hbm, rowv, colv, gbuf, acc_sh, sem):
    c = lax.axis_index("c")
    s = lax.axis_index("s")
    wid = c * NS + s
    zero = jnp.zeros((L,), jnp.float32)

    def zbody(r, carry):
        for j in range(D // L):
            gbuf[r, pl.ds(j * L, L)] = zero
        return carry

    lax.fori_loop(0, CHUNK, zbody, 0)
    for j in range(ZROWS // CHUNK):
        pltpu.sync_copy(gbuf, acc_sh.at[pl.ds(s * ZROWS + j * CHUNK, CHUNK)])

    pltpu.sync_copy(row_hbm.at[pl.ds(wid * K_T, K_T)], rowv)
    pltpu.sync_copy(col_hbm.at[pl.ds(wid * K_T, K_T)], colv)
    plsc.subcore_barrier()

    def body(k, carry):
        pltpu.async_copy(g_hbm.at[rowv.at[k]], gbuf, sem).wait()
        pltpu.sync_copy(gbuf, acc_sh.at[colv.at[k]], add=True)
        return carry

    lax.fori_loop(0, K_T, body, 0)
    plsc.subcore_barrier()
    pltpu.sync_copy(
        acc_sh.at[pl.ds(s * ZROWS, ZROWS)],
        out_hbm.at[c].at[pl.ds(s * ZROWS, ZROWS)],
    )


R = 1000
GRID = N // R


def _dinv_of(dp_ref):
    deg = jnp.sum(dp_ref[...], axis=0) + 1.0
    return lax.rsqrt(deg)


def _tc_prep_body(dp, x, w, g):
    dinv = _dinv_of(dp)
    h = jnp.dot(x[...], w[...], preferred_element_type=jnp.float32)
    g[...] = h * dinv[:, None]


_tc_prep = pl.pallas_call(
    _tc_prep_body,
    grid=(GRID,),
    in_specs=[
        pl.BlockSpec((NW, R), lambda i: (0, i)),
        pl.BlockSpec((R, D), lambda i: (i, 0)),
        pl.BlockSpec((D, D), lambda i: (0, 0)),
    ],
    out_specs=pl.BlockSpec((R, D), lambda i: (i, 0)),
    out_shape=jax.ShapeDtypeStruct((N, D), jnp.float32),
)


def _tc_mid_body(acc, g1, dp, w2, b1, g2ref):
    dinv = _dinv_of(dp)
    t = acc[0] + acc[1] + g1[...]
    h = jnp.maximum(t * dinv[:, None] + b1[...], 0.0)
    g2ref[...] = jnp.dot(h, w2[...], preferred_element_type=jnp.float32) * dinv[:, None]


_tc_mid = pl.pallas_call(
    _tc_mid_body,
    grid=(GRID,),
    in_specs=[
        pl.BlockSpec((NC, R, D), lambda i: (0, i, 0)),
        pl.BlockSpec((R, D), lambda i: (i, 0)),
        pl.BlockSpec((NW, R), lambda i: (0, i)),
        pl.BlockSpec((D, D), lambda i: (0, 0)),
        pl.BlockSpec((1, D), lambda i: (0, 0)),
    ],
    out_specs=pl.BlockSpec((R, D), lambda i: (i, 0)),
    out_shape=jax.ShapeDtypeStruct((N, D), jnp.float32),
)


def _tc_final_body(acc, g2, dp, b2, wfc, bfc, outref):
    dinv = _dinv_of(dp)
    t = acc[0] + acc[1] + g2[...]
    h = jnp.maximum(t * dinv[:, None] + b2[...], 0.0)
    outref[...] = jnp.dot(h, wfc[...], preferred_element_type=jnp.float32) + bfc[...]


_tc_final = pl.pallas_call(
    _tc_final_body,
    grid=(GRID,),
    in_specs=[
        pl.BlockSpec((NC, R, D), lambda i: (0, i, 0)),
        pl.BlockSpec((R, D), lambda i: (i, 0)),
        pl.BlockSpec((NW, R), lambda i: (0, i)),
        pl.BlockSpec((1, D), lambda i: (0, 0)),
        pl.BlockSpec((D, DO), lambda i: (0, 0)),
        pl.BlockSpec((1, DO), lambda i: (0, 0)),
    ],
    out_specs=pl.BlockSpec((R, DO), lambda i: (i, 0)),
    out_shape=jax.ShapeDtypeStruct((N, DO), jnp.float32),
)


def kernel(x, edge_index, W1, b1, W2, b2, Wfc, bfc):
    row = edge_index[0]
    col = edge_index[1]
    deg_parts = _sc_degree(col)

    pad = E_PAD - E
    rowp = jnp.concatenate([row, jnp.zeros((pad,), jnp.int32)]).reshape(
        NW * K_T, CHUNK
    )
    colp = jnp.concatenate([col, jnp.full((pad,), DUMMY, jnp.int32)]).reshape(
        NW * K_T, CHUNK
    )

    g1 = _tc_prep(deg_parts, x, W1)
    acc1 = _sc_msg(g1, rowp, colp)
    g2 = _tc_mid(acc1, g1, deg_parts, W2, b1.reshape(1, D))
    acc2 = _sc_msg(g2, rowp, colp)
    return _tc_final(acc2, g2, deg_parts, b2.reshape(1, D), Wfc, bfc.reshape(1, DO))


# speedup vs baseline: 8.9417x; 8.9417x over previous
import functools

import jax
import jax.numpy as jnp
from jax import lax
from jax.experimental import pallas as pl
from jax.experimental.pallas import tpu as pltpu
from jax.experimental.pallas import tpu_sc as plsc

N = 10000
D = 128
DO = 2
E = 320000

NC = 2
NS = 16
NW = NC * NS
L = 16

EH_T = E // NW

CHUNK = 128
K_T = 80
E_T = CHUNK * K_T
E_PAD = E_T * NW
N_PAD = 10240
ZROWS = N_PAD // NS
DUMMY = N_PAD - 1

_mesh = plsc.VectorSubcoreMesh(
    core_axis_name="c", subcore_axis_name="s", num_cores=NC, num_subcores=NS
)


@functools.partial(
    pl.kernel,
    out_type=jax.ShapeDtypeStruct((NW, N), jnp.float32),
    mesh=_mesh,
    scratch_types=[
        pltpu.VMEM((EH_T,), jnp.int32),
        pltpu.VMEM((N,), jnp.float32),
    ],
    compiler_params=pltpu.CompilerParams(needs_layout_passes=False),
)
def _sc_degree(col_hbm, deg_parts_hbm, colv, degv):
    c = lax.axis_index("c")
    s = lax.axis_index("s")
    wid = c * NS + s
    pltpu.sync_copy(col_hbm.at[pl.ds(wid * EH_T, EH_T)], colv)
    zero = jnp.zeros((L,), jnp.float32)

    def zbody(i, carry):
        degv[pl.ds(i * L, L)] = zero
        return carry

    lax.fori_loop(0, N // L, zbody, 0)
    ones = jnp.ones((L,), jnp.float32)

    def hbody(i, carry):
        idx = colv[pl.ds(i * L, L)]
        plsc.addupdate_scatter(degv, [idx], ones)
        return carry

    lax.fori_loop(0, EH_T // L, hbody, 0)
    pltpu.sync_copy(degv, deg_parts_hbm.at[wid])


@functools.partial(
    pl.kernel,
    out_type=jax.ShapeDtypeStruct((NC, N_PAD, D), jnp.float32),
    mesh=_mesh,
    scratch_types=[
        pltpu.VMEM((K_T, CHUNK), jnp.int32),
        pltpu.VMEM((K_T, CHUNK), jnp.int32),
        pltpu.VMEM((CHUNK, D), jnp.float32),
        pltpu.VMEM_SHARED((N_PAD, D), jnp.float32),
        pltpu.SemaphoreType.DMA,
    ],
    compiler_params=pltpu.CompilerParams(needs_layout_passes=False),
)
def _sc_msg(g_hbm, row_hbm, col_hbm, out_hbm, rowv, colv, gbuf, acc_sh, sem):
    c = lax.axis_index("c")
    s = lax.axis_index("s")
    wid = c * NS + s
    zero = jnp.zeros((L,), jnp.float32)

    def zbody(r, carry):
        for j in range(D // L):
            gbuf[r, pl.ds(j * L, L)] = zero
        return carry

    lax.fori_loop(0, CHUNK, zbody, 0)
    for j in range(ZROWS // CHUNK):
        pltpu.sync_copy(gbuf, acc_sh.at[pl.ds(s * ZROWS + j * CHUNK, CHUNK)])

    pltpu.sync_copy(row_hbm.at[pl.ds(wid * K_T, K_T)], rowv)
    pltpu.sync_copy(col_hbm.at[pl.ds(wid * K_T, K_T)], colv)
    plsc.subcore_barrier()

    def body(k, carry):
        pltpu.async_copy(g_hbm.at[rowv.at[k]], gbuf, sem).wait()
        pltpu.sync_copy(gbuf, acc_sh.at[colv.at[k]], add=True)
        return carry

    lax.fori_loop(0, K_T, body, 0)
    plsc.subcore_barrier()
    pltpu.sync_copy(
        acc_sh.at[pl.ds(s * ZROWS, ZROWS)],
        out_hbm.at[c].at[pl.ds(s * ZROWS, ZROWS)],
    )


R = 1000
GRID = N // R


def _dinv_of(dp_ref):
    deg = jnp.sum(dp_ref[...], axis=1, keepdims=True) + 1.0
    return lax.rsqrt(deg)


def _tc_prep_body(dp, x, w, g):
    dinv = _dinv_of(dp)
    h = jnp.dot(x[...], w[...], preferred_element_type=jnp.float32)
    g[...] = h * dinv


_tc_prep = pl.pallas_call(
    _tc_prep_body,
    grid=(GRID,),
    in_specs=[
        pl.BlockSpec((R, NW), lambda i: (i, 0)),
        pl.BlockSpec((R, D), lambda i: (i, 0)),
        pl.BlockSpec((D, D), lambda i: (0, 0)),
    ],
    out_specs=pl.BlockSpec((R, D), lambda i: (i, 0)),
    out_shape=jax.ShapeDtypeStruct((N, D), jnp.float32),
)


def _tc_mid_body(acc, g1, dp, w2, b1, g2ref):
    dinv = _dinv_of(dp)
    t = acc[0] + acc[1] + g1[...]
    h = jnp.maximum(t * dinv + b1[...], 0.0)
    g2ref[...] = jnp.dot(h, w2[...], preferred_element_type=jnp.float32) * dinv


_tc_mid = pl.pallas_call(
    _tc_mid_body,
    grid=(GRID,),
    in_specs=[
        pl.BlockSpec((NC, R, D), lambda i: (0, i, 0)),
        pl.BlockSpec((R, D), lambda i: (i, 0)),
        pl.BlockSpec((R, NW), lambda i: (i, 0)),
        pl.BlockSpec((D, D), lambda i: (0, 0)),
        pl.BlockSpec((1, D), lambda i: (0, 0)),
    ],
    out_specs=pl.BlockSpec((R, D), lambda i: (i, 0)),
    out_shape=jax.ShapeDtypeStruct((N, D), jnp.float32),
)


def _tc_final_body(acc, g2, dp, b2, wfc, bfc, outref):
    dinv = _dinv_of(dp)
    t = acc[0] + acc[1] + g2[...]
    h = jnp.maximum(t * dinv + b2[...], 0.0)
    outref[...] = jnp.dot(h, wfc[...], preferred_element_type=jnp.float32) + bfc[...]


_tc_final = pl.pallas_call(
    _tc_final_body,
    grid=(GRID,),
    in_specs=[
        pl.BlockSpec((NC, R, D), lambda i: (0, i, 0)),
        pl.BlockSpec((R, D), lambda i: (i, 0)),
        pl.BlockSpec((R, NW), lambda i: (i, 0)),
        pl.BlockSpec((1, D), lambda i: (0, 0)),
        pl.BlockSpec((D, DO), lambda i: (0, 0)),
        pl.BlockSpec((1, DO), lambda i: (0, 0)),
    ],
    out_specs=pl.BlockSpec((R, DO), lambda i: (i, 0)),
    out_shape=jax.ShapeDtypeStruct((N, DO), jnp.float32),
)


def kernel(x, edge_index, W1, b1, W2, b2, Wfc, bfc):
    row = edge_index[0]
    col = edge_index[1]
    deg_parts = _sc_degree(col).T

    pad = E_PAD - E
    rowp = jnp.concatenate([row, jnp.zeros((pad,), jnp.int32)]).reshape(
        NW * K_T, CHUNK
    )
    colp = jnp.concatenate([col, jnp.full((pad,), DUMMY, jnp.int32)]).reshape(
        NW * K_T, CHUNK
    )

    g1 = _tc_prep(deg_parts, x, W1)
    acc1 = _sc_msg(g1, rowp, colp)
    g2 = _tc_mid(acc1, g1, deg_parts, W2, b1.reshape(1, D))
    acc2 = _sc_msg(g2, rowp, colp)
    return _tc_final(acc2, g2, deg_parts, b2.reshape(1, D), Wfc, bfc.reshape(1, DO))


# speedup vs baseline: 9.7876x; 1.0946x over previous
import functools

import jax
import jax.numpy as jnp
from jax import lax
from jax.experimental import pallas as pl
from jax.experimental.pallas import tpu as pltpu
from jax.experimental.pallas import tpu_sc as plsc

N = 10000
D = 128
DO = 2
E = 320000

NC = 2
NS = 16
NW = NC * NS
L = 16

EH_T = E // NW

CHUNK = 128
K_T = 80
KH = K_T // 2
E_T = CHUNK * K_T
E_PAD = E_T * NW
N_PAD = 10112
ZROWS = N_PAD // NS
DUMMY = N_PAD - 1

_mesh = plsc.VectorSubcoreMesh(
    core_axis_name="c", subcore_axis_name="s", num_cores=NC, num_subcores=NS
)


@functools.partial(
    pl.kernel,
    out_type=jax.ShapeDtypeStruct((NW, N), jnp.float32),
    mesh=_mesh,
    scratch_types=[
        pltpu.VMEM((EH_T,), jnp.int32),
        pltpu.VMEM((N,), jnp.float32),
    ],
    compiler_params=pltpu.CompilerParams(needs_layout_passes=False),
)
def _sc_degree(col_hbm, deg_parts_hbm, colv, degv):
    c = lax.axis_index("c")
    s = lax.axis_index("s")
    wid = c * NS + s
    pltpu.sync_copy(col_hbm.at[pl.ds(wid * EH_T, EH_T)], colv)
    zero = jnp.zeros((L,), jnp.float32)

    def zbody(i, carry):
        degv[pl.ds(i * L, L)] = zero
        return carry

    lax.fori_loop(0, N // L, zbody, 0)
    ones = jnp.ones((L,), jnp.float32)

    def hbody(i, carry):
        idx = colv[pl.ds(i * L, L)]
        plsc.addupdate_scatter(degv, [idx], ones)
        return carry

    lax.fori_loop(0, EH_T // L, hbody, 0)
    pltpu.sync_copy(degv, deg_parts_hbm.at[wid])


@functools.partial(
    pl.kernel,
    out_type=jax.ShapeDtypeStruct((NC, N_PAD, D), jnp.float32),
    mesh=_mesh,
    scratch_types=[
        pltpu.VMEM((KH, CHUNK), jnp.int32),
        pltpu.VMEM((KH, CHUNK), jnp.int32),
        pltpu.VMEM((CHUNK, D), jnp.float32),
        pltpu.VMEM((CHUNK, D), jnp.float32),
        pltpu.VMEM_SHARED((N_PAD, D), jnp.float32),
        pltpu.SemaphoreType.DMA,
        pltpu.SemaphoreType.DMA,
    ],
    compiler_params=pltpu.CompilerParams(needs_layout_passes=False),
)
def _sc_msg(g_hbm, row_hbm, col_hbm, out_hbm, rowv, colv, gb0, gb1, acc_sh,
            sem0, sem1):
    c = lax.axis_index("c")
    s = lax.axis_index("s")
    wid = c * NS + s
    zero = jnp.zeros((L,), jnp.float32)

    def zbody(r, carry):
        for j in range(D // L):
            gb0[r, pl.ds(j * L, L)] = zero
        return carry

    lax.fori_loop(0, CHUNK, zbody, 0)
    for j in range(ZROWS // CHUNK):
        pltpu.sync_copy(gb0, acc_sh.at[pl.ds(s * ZROWS + j * CHUNK, CHUNK)])
    pltpu.sync_copy(
        gb0.at[pl.ds(0, ZROWS % CHUNK)],
        acc_sh.at[pl.ds(s * ZROWS + (ZROWS // CHUNK) * CHUNK, ZROWS % CHUNK)],
    )
    plsc.subcore_barrier()

    for h in range(K_T // KH):
        pltpu.sync_copy(row_hbm.at[pl.ds(wid * K_T + h * KH, KH)], rowv)
        pltpu.sync_copy(col_hbm.at[pl.ds(wid * K_T + h * KH, KH)], colv)
        pltpu.async_copy(g_hbm.at[rowv.at[0]], gb0, sem0)

        def body(i, carry):
            k0 = 2 * i
            pltpu.async_copy(g_hbm.at[rowv.at[k0 + 1]], gb1, sem1)
            pltpu.make_async_copy(g_hbm.at[rowv.at[k0]], gb0, sem0).wait()
            pltpu.sync_copy(gb0, acc_sh.at[colv.at[k0]], add=True)

            @pl.when(i < KH // 2 - 1)
            def _():
                pltpu.async_copy(g_hbm.at[rowv.at[k0 + 2]], gb0, sem0)

            pltpu.make_async_copy(g_hbm.at[rowv.at[k0 + 1]], gb1, sem1).wait()
            pltpu.sync_copy(gb1, acc_sh.at[colv.at[k0 + 1]], add=True)
            return carry

        lax.fori_loop(0, KH // 2, body, 0)
    plsc.subcore_barrier()
    pltpu.sync_copy(
        acc_sh.at[pl.ds(s * ZROWS, ZROWS)],
        out_hbm.at[c].at[pl.ds(s * ZROWS, ZROWS)],
    )


R = 1000
GRID = N // R


def _dinv_of(dp_ref):
    deg = jnp.sum(dp_ref[...], axis=1, keepdims=True) + 1.0
    return lax.rsqrt(deg)


def _tc_prep_body(dp, x, w, g):
    dinv = _dinv_of(dp)
    h = jnp.dot(x[...], w[...], preferred_element_type=jnp.float32)
    g[...] = h * dinv


_tc_prep = pl.pallas_call(
    _tc_prep_body,
    grid=(GRID,),
    in_specs=[
        pl.BlockSpec((R, NW), lambda i: (i, 0)),
        pl.BlockSpec((R, D), lambda i: (i, 0)),
        pl.BlockSpec((D, D), lambda i: (0, 0)),
    ],
    out_specs=pl.BlockSpec((R, D), lambda i: (i, 0)),
    out_shape=jax.ShapeDtypeStruct((N, D), jnp.float32),
)


def _tc_mid_body(acc, g1, dp, w2, b1, g2ref):
    dinv = _dinv_of(dp)
    t = acc[0] + acc[1] + g1[...]
    h = jnp.maximum(t * dinv + b1[...], 0.0)
    g2ref[...] = jnp.dot(h, w2[...], preferred_element_type=jnp.float32) * dinv


_tc_mid = pl.pallas_call(
    _tc_mid_body,
    grid=(GRID,),
    in_specs=[
        pl.BlockSpec((NC, R, D), lambda i: (0, i, 0)),
        pl.BlockSpec((R, D), lambda i: (i, 0)),
        pl.BlockSpec((R, NW), lambda i: (i, 0)),
        pl.BlockSpec((D, D), lambda i: (0, 0)),
        pl.BlockSpec((1, D), lambda i: (0, 0)),
    ],
    out_specs=pl.BlockSpec((R, D), lambda i: (i, 0)),
    out_shape=jax.ShapeDtypeStruct((N, D), jnp.float32),
)


def _tc_final_body(acc, g2, dp, b2, wfc, bfc, outref):
    dinv = _dinv_of(dp)
    t = acc[0] + acc[1] + g2[...]
    h = jnp.maximum(t * dinv + b2[...], 0.0)
    outref[...] = jnp.dot(h, wfc[...], preferred_element_type=jnp.float32) + bfc[...]


_tc_final = pl.pallas_call(
    _tc_final_body,
    grid=(GRID,),
    in_specs=[
        pl.BlockSpec((NC, R, D), lambda i: (0, i, 0)),
        pl.BlockSpec((R, D), lambda i: (i, 0)),
        pl.BlockSpec((R, NW), lambda i: (i, 0)),
        pl.BlockSpec((1, D), lambda i: (0, 0)),
        pl.BlockSpec((D, DO), lambda i: (0, 0)),
        pl.BlockSpec((1, DO), lambda i: (0, 0)),
    ],
    out_specs=pl.BlockSpec((R, DO), lambda i: (i, 0)),
    out_shape=jax.ShapeDtypeStruct((N, DO), jnp.float32),
)


def kernel(x, edge_index, W1, b1, W2, b2, Wfc, bfc):
    row = edge_index[0]
    col = edge_index[1]
    deg_parts = _sc_degree(col).T

    pad = E_PAD - E
    rowp = jnp.concatenate([row, jnp.zeros((pad,), jnp.int32)]).reshape(
        NW * K_T, CHUNK
    )
    colp = jnp.concatenate([col, jnp.full((pad,), DUMMY, jnp.int32)]).reshape(
        NW * K_T, CHUNK
    )

    g1 = _tc_prep(deg_parts, x, W1)
    acc1 = _sc_msg(g1, rowp, colp)
    g2 = _tc_mid(acc1, g1, deg_parts, W2, b1.reshape(1, D))
    acc2 = _sc_msg(g2, rowp, colp)
    return _tc_final(acc2, g2, deg_parts, b2.reshape(1, D), Wfc, bfc.reshape(1, DO))


# speedup vs baseline: 11.4483x; 1.1697x over previous
import functools

import jax
import jax.numpy as jnp
from jax import lax
from jax.experimental import pallas as pl
from jax.experimental.pallas import tpu as pltpu
from jax.experimental.pallas import tpu_sc as plsc

N = 10000
D = 128
DO = 2
E = 320000

NC = 2
NS = 16
NW = NC * NS
L = 16

EH_T = E // NW

CHUNK = 128
K_T = 80
KH = K_T // 2
E_T = CHUNK * K_T
E_PAD = E_T * NW
N_PAD = 10240
ZROWS = N_PAD // NS
PAD_T = E_T - EH_T

_mesh = plsc.VectorSubcoreMesh(
    core_axis_name="c", subcore_axis_name="s", num_cores=NC, num_subcores=NS
)


@functools.partial(
    pl.kernel,
    out_type=jax.ShapeDtypeStruct((NW, N), jnp.float32),
    mesh=_mesh,
    scratch_types=[
        pltpu.VMEM((EH_T,), jnp.int32),
        pltpu.VMEM((N,), jnp.float32),
    ],
    compiler_params=pltpu.CompilerParams(needs_layout_passes=False),
)
def _sc_degree(col_hbm, deg_parts_hbm, colv, degv):
    c = lax.axis_index("c")
    s = lax.axis_index("s")
    wid = c * NS + s
    pltpu.sync_copy(col_hbm.at[pl.ds(wid * EH_T, EH_T)], colv)
    zero = jnp.zeros((L,), jnp.float32)

    def zbody(i, carry):
        degv[pl.ds(i * L, L)] = zero
        return carry

    lax.fori_loop(0, N // L, zbody, 0)
    ones = jnp.ones((L,), jnp.float32)

    def hbody(i, carry):
        idx = colv[pl.ds(i * L, L)]
        plsc.addupdate_scatter(degv, [idx], ones)
        return carry

    lax.fori_loop(0, EH_T // L, hbody, 0)
    pltpu.sync_copy(degv, deg_parts_hbm.at[wid])


@functools.partial(
    pl.kernel,
    out_type=jax.ShapeDtypeStruct((NC, N_PAD, D), jnp.float32),
    mesh=_mesh,
    scratch_types=[
        pltpu.VMEM((KH, CHUNK), jnp.int32),
        pltpu.VMEM((KH, CHUNK), jnp.int32),
        pltpu.VMEM((CHUNK, D), jnp.float32),
        pltpu.VMEM((CHUNK, D), jnp.float32),
        pltpu.VMEM_SHARED((N_PAD, D), jnp.float32),
        pltpu.SemaphoreType.DMA,
        pltpu.SemaphoreType.DMA,
    ],
    compiler_params=pltpu.CompilerParams(needs_layout_passes=False),
)
def _sc_msg(g_hbm, row_hbm, col_hbm, out_hbm, rowv, colv, gb0, gb1, acc_sh,
            sem0, sem1):
    c = lax.axis_index("c")
    s = lax.axis_index("s")
    wid = c * NS + s
    zero = jnp.zeros((L,), jnp.float32)

    def zbody(r, carry):
        for j in range(D // L):
            gb0[r, pl.ds(j * L, L)] = zero
        return carry

    lax.fori_loop(0, CHUNK, zbody, 0)
    for j in range(ZROWS // CHUNK):
        pltpu.sync_copy(gb0, acc_sh.at[pl.ds(s * ZROWS + j * CHUNK, CHUNK)])
    plsc.subcore_barrier()

    for h in range(K_T // KH):
        pltpu.sync_copy(row_hbm.at[pl.ds(wid * K_T + h * KH, KH)], rowv)
        pltpu.sync_copy(col_hbm.at[pl.ds(wid * K_T + h * KH, KH)], colv)
        pltpu.async_copy(g_hbm.at[rowv.at[0]], gb0, sem0)

        def body(i, carry):
            k0 = 2 * i
            pltpu.async_copy(g_hbm.at[rowv.at[k0 + 1]], gb1, sem1)
            pltpu.make_async_copy(g_hbm.at[rowv.at[k0]], gb0, sem0).wait()
            pltpu.sync_copy(gb0, acc_sh.at[colv.at[k0]], add=True)

            @pl.when(i < KH // 2 - 1)
            def _():
                pltpu.async_copy(g_hbm.at[rowv.at[k0 + 2]], gb0, sem0)

            pltpu.make_async_copy(g_hbm.at[rowv.at[k0 + 1]], gb1, sem1).wait()
            pltpu.sync_copy(gb1, acc_sh.at[colv.at[k0 + 1]], add=True)
            return carry

        lax.fori_loop(0, KH // 2, body, 0)
    plsc.subcore_barrier()
    pltpu.sync_copy(
        acc_sh.at[pl.ds(s * ZROWS, ZROWS)],
        out_hbm.at[c].at[pl.ds(s * ZROWS, ZROWS)],
    )


R = 1000
GRID = N // R


def _dinv_of(dp_ref):
    deg = jnp.sum(dp_ref[...], axis=1, keepdims=True) + 1.0
    return lax.rsqrt(deg)


def _tc_prep_body(dp, x, w, g):
    dinv = _dinv_of(dp)
    h = jnp.dot(x[...], w[...], preferred_element_type=jnp.float32)
    g[...] = h * dinv


_tc_prep = pl.pallas_call(
    _tc_prep_body,
    grid=(GRID,),
    in_specs=[
        pl.BlockSpec((R, NW), lambda i: (i, 0)),
        pl.BlockSpec((R, D), lambda i: (i, 0)),
        pl.BlockSpec((D, D), lambda i: (0, 0)),
    ],
    out_specs=pl.BlockSpec((R, D), lambda i: (i, 0)),
    out_shape=jax.ShapeDtypeStruct((N, D), jnp.float32),
)


def _tc_mid_body(acc, g1, dp, w2, b1, g2ref):
    dinv = _dinv_of(dp)
    t = acc[0] + acc[1] + g1[...]
    h = jnp.maximum(t * dinv + b1[...], 0.0)
    g2ref[...] = jnp.dot(h, w2[...], preferred_element_type=jnp.float32) * dinv


_tc_mid = pl.pallas_call(
    _tc_mid_body,
    grid=(GRID,),
    in_specs=[
        pl.BlockSpec((NC, R, D), lambda i: (0, i, 0)),
        pl.BlockSpec((R, D), lambda i: (i, 0)),
        pl.BlockSpec((R, NW), lambda i: (i, 0)),
        pl.BlockSpec((D, D), lambda i: (0, 0)),
        pl.BlockSpec((1, D), lambda i: (0, 0)),
    ],
    out_specs=pl.BlockSpec((R, D), lambda i: (i, 0)),
    out_shape=jax.ShapeDtypeStruct((N, D), jnp.float32),
)


def _tc_final_body(acc, g2, dp, b2, wfc, bfc, outref):
    dinv = _dinv_of(dp)
    t = acc[0] + acc[1] + g2[...]
    h = jnp.maximum(t * dinv + b2[...], 0.0)
    outref[...] = jnp.dot(h, wfc[...], preferred_element_type=jnp.float32) + bfc[...]


_tc_final = pl.pallas_call(
    _tc_final_body,
    grid=(GRID,),
    in_specs=[
        pl.BlockSpec((NC, R, D), lambda i: (0, i, 0)),
        pl.BlockSpec((R, D), lambda i: (i, 0)),
        pl.BlockSpec((R, NW), lambda i: (i, 0)),
        pl.BlockSpec((1, D), lambda i: (0, 0)),
        pl.BlockSpec((D, DO), lambda i: (0, 0)),
        pl.BlockSpec((1, DO), lambda i: (0, 0)),
    ],
    out_specs=pl.BlockSpec((R, DO), lambda i: (i, 0)),
    out_shape=jax.ShapeDtypeStruct((N, DO), jnp.float32),
)


def kernel(x, edge_index, W1, b1, W2, b2, Wfc, bfc):
    row = edge_index[0]
    col = edge_index[1]
    deg_parts = _sc_degree(col).T

    rowp = jnp.concatenate(
        [row.reshape(NW, EH_T), jnp.zeros((NW, PAD_T), jnp.int32)], axis=1
    ).reshape(NW * K_T, CHUNK)
    pad_dst = jnp.broadcast_to(N + jnp.arange(PAD_T, dtype=jnp.int32), (NW, PAD_T))
    colp = jnp.concatenate([col.reshape(NW, EH_T), pad_dst], axis=1).reshape(
        NW * K_T, CHUNK
    )

    g1 = _tc_prep(deg_parts, x, W1)
    acc1 = _sc_msg(g1, rowp, colp)
    g2 = _tc_mid(acc1, g1, deg_parts, W2, b1.reshape(1, D))
    acc2 = _sc_msg(g2, rowp, colp)
    return _tc_final(acc2, g2, deg_parts, b2.reshape(1, D), Wfc, bfc.reshape(1, DO))


# speedup vs baseline: 33.0119x; 2.8836x over previous
import functools

import jax
import jax.numpy as jnp
from jax import lax
from jax.experimental import pallas as pl
from jax.experimental.pallas import tpu as pltpu
from jax.experimental.pallas import tpu_sc as plsc

N = 10000
D = 128
DO = 2
E = 320000

NC = 2
NS = 16
NW = NC * NS
L = 16

EH_T = E // NW

CHUNK = 128
K_T = 80
KH = K_T // 2
E_T = CHUNK * K_T
E_PAD = E_T * NW
N_PAD = 10240
ZROWS = N_PAD // NS
PAD_T = E_T - EH_T

_mesh = plsc.VectorSubcoreMesh(
    core_axis_name="c", subcore_axis_name="s", num_cores=NC, num_subcores=NS
)


@functools.partial(
    pl.kernel,
    out_type=jax.ShapeDtypeStruct((NW, N), jnp.float32),
    mesh=_mesh,
    scratch_types=[
        pltpu.VMEM((EH_T,), jnp.int32),
        pltpu.VMEM((N,), jnp.float32),
    ],
    compiler_params=pltpu.CompilerParams(needs_layout_passes=False),
)
def _sc_degree(col_hbm, deg_parts_hbm, colv, degv):
    c = lax.axis_index("c")
    s = lax.axis_index("s")
    wid = c * NS + s
    pltpu.sync_copy(col_hbm.at[pl.ds(wid * EH_T, EH_T)], colv)
    zero = jnp.zeros((L,), jnp.float32)

    def zbody(i, carry):
        degv[pl.ds(i * L, L)] = zero
        return carry

    lax.fori_loop(0, N // L, zbody, 0)
    ones = jnp.ones((L,), jnp.float32)

    def hbody(i, carry):
        idx = colv[pl.ds(i * L, L)]
        plsc.addupdate_scatter(degv, [idx], ones)
        return carry

    lax.fori_loop(0, EH_T // L, hbody, 0)
    pltpu.sync_copy(degv, deg_parts_hbm.at[wid])


@functools.partial(
    pl.kernel,
    out_type=jax.ShapeDtypeStruct((NC, N_PAD, D), jnp.float32),
    mesh=_mesh,
    scratch_types=[
        pltpu.VMEM((KH, CHUNK), jnp.int32),
        pltpu.VMEM((KH, CHUNK), jnp.int32),
        pltpu.VMEM((CHUNK, D), jnp.float32),
        pltpu.VMEM((CHUNK, D), jnp.float32),
        pltpu.VMEM_SHARED((N_PAD, D), jnp.float32),
        pltpu.SemaphoreType.DMA,
        pltpu.SemaphoreType.DMA,
    ],
    compiler_params=pltpu.CompilerParams(needs_layout_passes=False),
)
def _sc_msg(g_hbm, row_hbm, col_hbm, out_hbm, rowv, colv, gb0, gb1, acc_sh,
            sem0, sem1):
    c = lax.axis_index("c")
    s = lax.axis_index("s")
    wid = c * NS + s
    zero = jnp.zeros((L,), jnp.float32)

    def zbody(r, carry):
        for j in range(D // L):
            gb0[r, pl.ds(j * L, L)] = zero
        return carry

    lax.fori_loop(0, CHUNK, zbody, 0)
    for j in range(ZROWS // CHUNK):
        pltpu.sync_copy(gb0, acc_sh.at[pl.ds(s * ZROWS + j * CHUNK, CHUNK)])
    plsc.subcore_barrier()

    for h in range(K_T // KH):
        pltpu.sync_copy(row_hbm.at[pl.ds(wid * K_T + h * KH, KH)], rowv)
        pltpu.sync_copy(col_hbm.at[pl.ds(wid * K_T + h * KH, KH)], colv)
        pltpu.async_copy(g_hbm.at[rowv.at[0]], gb0, sem0)

        def body(i, carry):
            k0 = 2 * i
            pltpu.async_copy(g_hbm.at[rowv.at[k0 + 1]], gb1, sem1)
            pltpu.make_async_copy(g_hbm.at[rowv.at[k0]], gb0, sem0).wait()
            pltpu.sync_copy(gb0, acc_sh.at[colv.at[k0]], add=True)

            @pl.when(i < KH // 2 - 1)
            def _():
                pltpu.async_copy(g_hbm.at[rowv.at[k0 + 2]], gb0, sem0)

            pltpu.make_async_copy(g_hbm.at[rowv.at[k0 + 1]], gb1, sem1).wait()
            pltpu.sync_copy(gb1, acc_sh.at[colv.at[k0 + 1]], add=True)
            return carry

        lax.fori_loop(0, KH // 2, body, 0)
    plsc.subcore_barrier()
    pltpu.sync_copy(
        acc_sh.at[pl.ds(s * ZROWS, ZROWS)],
        out_hbm.at[c].at[pl.ds(s * ZROWS, ZROWS)],
    )


R = 1000
GRID = N // R


def _dinv_of(dp_ref):
    deg = jnp.sum(dp_ref[...], axis=1, keepdims=True) + 1.0
    return lax.rsqrt(deg)


def _tc_prep_body(dp, x, w, g):
    dinv = _dinv_of(dp)
    h = jnp.dot(x[...], w[...], preferred_element_type=jnp.float32)
    g[...] = h * dinv


_tc_prep = pl.pallas_call(
    _tc_prep_body,
    grid=(GRID,),
    in_specs=[
        pl.BlockSpec((R, NW), lambda i: (i, 0)),
        pl.BlockSpec((R, D), lambda i: (i, 0)),
        pl.BlockSpec((D, D), lambda i: (0, 0)),
    ],
    out_specs=pl.BlockSpec((R, D), lambda i: (i, 0)),
    out_shape=jax.ShapeDtypeStruct((N, D), jnp.float32),
)


def _tc_mid_body(acc, g1, dp, w2, b1, g2ref):
    dinv = _dinv_of(dp)
    t = acc[0] + acc[1] + g1[...]
    h = jnp.maximum(t * dinv + b1[...], 0.0)
    g2ref[...] = jnp.dot(h, w2[...], preferred_element_type=jnp.float32) * dinv


_tc_mid = pl.pallas_call(
    _tc_mid_body,
    grid=(GRID,),
    in_specs=[
        pl.BlockSpec((NC, R, D), lambda i: (0, i, 0)),
        pl.BlockSpec((R, D), lambda i: (i, 0)),
        pl.BlockSpec((R, NW), lambda i: (i, 0)),
        pl.BlockSpec((D, D), lambda i: (0, 0)),
        pl.BlockSpec((1, D), lambda i: (0, 0)),
    ],
    out_specs=pl.BlockSpec((R, D), lambda i: (i, 0)),
    out_shape=jax.ShapeDtypeStruct((N, D), jnp.float32),
)


def _tc_final_body(acc, g2, dp, b2, wfc, bfc, outref):
    dinv = _dinv_of(dp)
    t = acc[0] + acc[1] + g2[...]
    h = jnp.maximum(t * dinv + b2[...], 0.0)
    outref[...] = jnp.dot(h, wfc[...], preferred_element_type=jnp.float32) + bfc[...]


_tc_final = pl.pallas_call(
    _tc_final_body,
    grid=(GRID,),
    in_specs=[
        pl.BlockSpec((NC, R, D), lambda i: (0, i, 0)),
        pl.BlockSpec((R, D), lambda i: (i, 0)),
        pl.BlockSpec((R, NW), lambda i: (i, 0)),
        pl.BlockSpec((1, D), lambda i: (0, 0)),
        pl.BlockSpec((D, DO), lambda i: (0, 0)),
        pl.BlockSpec((1, DO), lambda i: (0, 0)),
    ],
    out_specs=pl.BlockSpec((R, DO), lambda i: (i, 0)),
    out_shape=jax.ShapeDtypeStruct((N, DO), jnp.float32),
)


def kernel(x, edge_index, W1, b1, W2, b2, Wfc, bfc):
    row = edge_index[0]
    col = edge_index[1]
    deg_parts = _sc_degree(col).T

    pad_src = (
        jnp.arange(NW, dtype=jnp.int32)[:, None] * PAD_T
        + jnp.arange(PAD_T, dtype=jnp.int32)[None, :]
    ) % N
    rowp = jnp.concatenate(
        [row.reshape(NW, EH_T), pad_src], axis=1
    ).reshape(NW * K_T, CHUNK)
    pad_dst = jnp.broadcast_to(N + jnp.arange(PAD_T, dtype=jnp.int32), (NW, PAD_T))
    colp = jnp.concatenate([col.reshape(NW, EH_T), pad_dst], axis=1).reshape(
        NW * K_T, CHUNK
    )

    g1 = _tc_prep(deg_parts, x, W1)
    acc1 = _sc_msg(g1, rowp, colp)
    g2 = _tc_mid(acc1, g1, deg_parts, W2, b1.reshape(1, D))
    acc2 = _sc_msg(g2, rowp, colp)
    return _tc_final(acc2, g2, deg_parts, b2.reshape(1, D), Wfc, bfc.reshape(1, DO))
